# baseline (speedup 1.0000x reference)
import jax
import jax.numpy as jnp
from jax.experimental import pallas as pl
from jax.experimental.pallas import tpu as pltpu

B, CIN, S, T = 32, 1, 128, 64
O, N = 1, 2048
STEP, LEAK = 16, 32
KS = STEP + LEAK
PAD = 32
FODEP = KS
THETA = 0.1 * (S * CIN)
SUP = 6.0
TP = T + 2 * PAD - KS + 1
TPAD = 128
TOFF = KS - 1 - PAD
NT = 512
NT2 = 256
NTILES = N // NT
BIGI = 1 << 20


def _phase1(x_ref, w_ref, lab_ref, m_ref, a_ref, pot_ref):
    i = pl.program_id(0)
    w = jnp.transpose(w_ref[...], (1, 0))
    xt = jnp.transpose(x_ref[...], (0, 2, 1)).reshape(B * T, S)
    xb = xt.astype(jnp.bfloat16)
    xb3 = jnp.concatenate([xb, xb, xb], axis=1)
    for k in range(KS):
        tj = jnp.float32(KS - 1 - k)
        t_spike = tj / STEP
        t_leak = -(tj - w * STEP) / LEAK + w
        wk = jnp.maximum(0.0, jnp.minimum(t_spike, t_leak))
        hi = wk.astype(jnp.bfloat16)
        r1 = wk - hi.astype(jnp.float32)
        mid = r1.astype(jnp.bfloat16)
        lo = (r1 - mid.astype(jnp.float32)).astype(jnp.bfloat16)
        wk3 = jnp.concatenate([hi, mid, lo], axis=0)
        y = jax.lax.dot_general(xb3, wk3, (((1,), (0,)), ((), ())),
                                preferred_element_type=jnp.float32)
        off = KS - 1 - k
        if k == 0:
            pot_ref[:, 0:off, :] = jnp.zeros((B, off, NT), jnp.float32)
            pot_ref[:, off + T:, :] = jnp.zeros((B, TPAD - off - T, NT), jnp.float32)
            pot_ref[:, off:off + T, :] = y.reshape(B, T, NT)
        else:
            pot_ref[:, off:off + T, :] += y.reshape(B, T, NT)
    nid = jax.lax.broadcasted_iota(jnp.int32, (B, NT), 1) + i * NT
    supm = nid == lab_ref[...]
    pot = pot_ref[...] + jnp.where(supm, SUP, 0.0)[:, None, :]
    tile_max = jnp.max(pot, axis=2)
    ids = jax.lax.broadcasted_iota(jnp.int32, (B, TPAD, NT), 2) + i * NT
    tile_arg = jnp.min(jnp.where(pot == tile_max[..., None], ids, BIGI), axis=2)

    @pl.when(i == 0)
    def _init():
        m_ref[...] = tile_max
        a_ref[...] = tile_arg

    @pl.when(i > 0)
    def _update():
        better = tile_max > m_ref[...]
        a_ref[...] = jnp.where(better, tile_arg, a_ref[...])
        m_ref[...] = jnp.where(better, tile_max, m_ref[...])


def _phase2(m_ref, a_ref, o_ref):
    i = pl.program_id(0)
    m = m_ref[...]
    a = a_ref[...]
    idx = jax.lax.broadcasted_iota(jnp.int32, (B, TPAD), 1)
    valid = (idx >= TOFF) & (idx < TOFF + TP)
    q = (m > THETA) & valid
    cand = jnp.where(q, idx, BIGI)
    s1 = jnp.min(cand, axis=1, keepdims=True)
    cand2 = jnp.where(q & (idx >= s1 + FODEP), idx, BIGI)
    s2 = jnp.min(cand2, axis=1, keepdims=True)
    spike = (idx == s1) | (idx == s2)
    wsel = jnp.where(spike, a, jnp.int32(-1))
    wsel = wsel[:, TOFF:TOFF + TP]
    nid = jax.lax.broadcasted_iota(jnp.int32, (B, NT2, TP), 1) + i * NT2
    o_ref[...] = (nid == wsel[:, None, :]).astype(jnp.float32)


def kernel(input_spikes, weight, labels):
    x = input_spikes.reshape(B, CIN * S, T)
    lab = labels.reshape(B, 1)

    m, a = pl.pallas_call(
        _phase1,
        grid=(NTILES,),
        in_specs=[
            pl.BlockSpec((B, S, T), lambda i: (0, 0, 0)),
            pl.BlockSpec((NT, S), lambda i: (i, 0)),
            pl.BlockSpec((B, 1), lambda i: (0, 0)),
        ],
        out_specs=[
            pl.BlockSpec((B, TPAD), lambda i: (0, 0)),
            pl.BlockSpec((B, TPAD), lambda i: (0, 0)),
        ],
        out_shape=[
            jax.ShapeDtypeStruct((B, TPAD), jnp.float32),
            jax.ShapeDtypeStruct((B, TPAD), jnp.int32),
        ],
        scratch_shapes=[pltpu.VMEM((B, TPAD, NT), jnp.float32)],
    )(x, weight, lab)

    out = pl.pallas_call(
        _phase2,
        grid=(N // NT2,),
        in_specs=[
            pl.BlockSpec((B, TPAD), lambda i: (0, 0)),
            pl.BlockSpec((B, TPAD), lambda i: (0, 0)),
        ],
        out_specs=pl.BlockSpec((B, NT2, TP), lambda i: (0, i, 0)),
        out_shape=jax.ShapeDtypeStruct((B, N, TP), jnp.float32),
    )(m, a)
    return out.reshape(B, O, N, TP)


# speedup vs baseline: 2.4435x; 2.4435x over previous
import jax
import jax.numpy as jnp
from jax.experimental import pallas as pl
from jax.experimental.pallas import tpu as pltpu

B, CIN, S, T = 32, 1, 128, 64
O, N = 1, 2048
STEP, LEAK = 16, 32
KS = STEP + LEAK
PAD = 32
FODEP = KS
THETA = 0.1 * (S * CIN)
SUP = 6.0
TP = T + 2 * PAD - KS + 1
TPAD = 128
TOFF = KS - 1 - PAD
NT = 512
NT2 = 256
NTILES = N // NT
BIGI = 1 << 20
RG = 8
NG = KS // RG
JX = 80


def _phase1(x_ref, w_ref, lab_ref, m_ref, a_ref, pot_ref, xs_ref):
    i = pl.program_id(0)

    @pl.when(i == 0)
    def _build_xs():
        xt = jnp.transpose(x_ref[...], (0, 2, 1))
        xs_ref[...] = jnp.zeros_like(xs_ref)
        for r in range(RG):
            xs_ref[:, 7 - r:7 - r + T, r * S:(r + 1) * S] = xt

    w = jnp.transpose(w_ref[...], (1, 0))
    wstack = jnp.concatenate([w] * RG, axis=0)
    rrow = (jax.lax.broadcasted_iota(jnp.int32, (RG * S, NT), 0)
            // S).astype(jnp.float32)
    xs = xs_ref[...].reshape(B * JX, RG * S)
    for g in range(NG):
        kg = g * RG
        tj = jnp.float32(KS - 1 - kg) - rrow
        t_spike = tj / STEP
        t_leak = -(tj - wstack * STEP) / LEAK + wstack
        wg = jnp.maximum(0.0, jnp.minimum(t_spike, t_leak))
        y = jax.lax.dot_general(xs, wg, (((1,), (0,)), ((), ())),
                                preferred_element_type=jnp.float32)
        lo = 40 - RG * g
        if g == 0:
            pot_ref[:, 0:lo, :] = jnp.zeros((B, lo, NT), jnp.float32)
            pot_ref[:, lo + JX:, :] = jnp.zeros((B, TPAD - lo - JX, NT), jnp.float32)
            pot_ref[:, lo:lo + JX, :] = y.reshape(B, JX, NT)
        else:
            pot_ref[:, lo:lo + JX, :] += y.reshape(B, JX, NT)
    nid = jax.lax.broadcasted_iota(jnp.int32, (B, NT), 1) + i * NT
    supm = nid == lab_ref[...]
    pot = pot_ref[...] + jnp.where(supm, SUP, 0.0)[:, None, :]
    tile_max = jnp.max(pot, axis=2)
    ids = jax.lax.broadcasted_iota(jnp.int32, (B, TPAD, NT), 2) + i * NT
    tile_arg = jnp.min(jnp.where(pot == tile_max[..., None], ids, BIGI), axis=2)

    @pl.when(i == 0)
    def _init():
        m_ref[...] = tile_max
        a_ref[...] = tile_arg

    @pl.when(i > 0)
    def _update():
        better = tile_max > m_ref[...]
        a_ref[...] = jnp.where(better, tile_arg, a_ref[...])
        m_ref[...] = jnp.where(better, tile_max, m_ref[...])


def _phase2(m_ref, a_ref, o_ref):
    i = pl.program_id(0)
    m = m_ref[...]
    a = a_ref[...]
    idx = jax.lax.broadcasted_iota(jnp.int32, (B, TPAD), 1)
    valid = (idx >= TOFF) & (idx < TOFF + TP)
    q = (m > THETA) & valid
    cand = jnp.where(q, idx, BIGI)
    s1 = jnp.min(cand, axis=1, keepdims=True)
    cand2 = jnp.where(q & (idx >= s1 + FODEP), idx, BIGI)
    s2 = jnp.min(cand2, axis=1, keepdims=True)
    spike = (idx == s1) | (idx == s2)
    wsel = jnp.where(spike, a, jnp.int32(-1))
    wsel = wsel[:, TOFF:TOFF + TP]
    nid = jax.lax.broadcasted_iota(jnp.int32, (B, NT2, TP), 1) + i * NT2
    o_ref[...] = (nid == wsel[:, None, :]).astype(jnp.float32)


def kernel(input_spikes, weight, labels):
    x = input_spikes.reshape(B, CIN * S, T)
    lab = labels.reshape(B, 1)

    m, a = pl.pallas_call(
        _phase1,
        grid=(NTILES,),
        in_specs=[
            pl.BlockSpec((B, S, T), lambda i: (0, 0, 0)),
            pl.BlockSpec((NT, S), lambda i: (i, 0)),
            pl.BlockSpec((B, 1), lambda i: (0, 0)),
        ],
        out_specs=[
            pl.BlockSpec((B, TPAD), lambda i: (0, 0)),
            pl.BlockSpec((B, TPAD), lambda i: (0, 0)),
        ],
        out_shape=[
            jax.ShapeDtypeStruct((B, TPAD), jnp.float32),
            jax.ShapeDtypeStruct((B, TPAD), jnp.int32),
        ],
        scratch_shapes=[pltpu.VMEM((B, TPAD, NT), jnp.float32),
                        pltpu.VMEM((B, JX, RG * S), jnp.float32)],
    )(x, weight, lab)

    out = pl.pallas_call(
        _phase2,
        grid=(N // NT2,),
        in_specs=[
            pl.BlockSpec((B, TPAD), lambda i: (0, 0)),
            pl.BlockSpec((B, TPAD), lambda i: (0, 0)),
        ],
        out_specs=pl.BlockSpec((B, NT2, TP), lambda i: (0, i, 0)),
        out_shape=jax.ShapeDtypeStruct((B, N, TP), jnp.float32),
    )(m, a)
    return out.reshape(B, O, N, TP)
